# baseline (device time: 22244 ns/iter reference)
import jax
import jax.numpy as jnp
from jax import lax
from jax.experimental import pallas as pl
from jax.experimental.pallas import tpu as pltpu

N_DEV = 16
N_SEG = 2


def _ring(t):
    t = lax.rem(t + 2 * N_DEV, N_DEV)
    q = t // 4
    j = t % 4
    return jnp.where(
        q == 0, 4 * j,
        jnp.where(q == 1, 15 - 4 * j,
                  jnp.where(q == 2, 4 * j + 2, 13 - 4 * j)),
    )


def _pos(p):
    z = p // 4
    c = p % 4
    return jnp.where(
        c == 0, z,
        jnp.where(c == 1, 15 - z, jnp.where(c == 2, 8 + z, 7 - z)),
    )


_SCHEDULE = (
    ((None, ((0, 8, "C"), (0, 4, "P"), (0, 12, "M"),
             (0, 1, "R"), (0, 9, "L"))),),
    ((1, ((1, 2, "R"),)), (9, ((9, 10, "L"),))),
    ((8, ((8, 7, "L"), (8, 15, "R"))),),
    ((4, ((4, 5, "R"),)), (12, ((12, 13, "L"),))),
    ((2, ((2, 3, "R"),)), (10, ((10, 11, "L"),))),
    ((7, ((7, 6, "L"),)), (15, ((15, 14, "R"),))),
    ((5, ()), (13, ()), (3, ()), (11, ()), (6, ()), (14, ())),
)


def kernel(x, w_mat):
    m_per, k = x.shape
    _, n_per = w_mat.shape
    m_seg = m_per // N_SEG

    def body(x_ref, w_ref, out_ref, gather_ref, w_bf, send_sems, recv_sems):
        my = lax.axis_index("i")
        r = _pos(my)
        left = _ring(r - 1)
        right = _ring(r + 1)
        chord = _ring(r + 8)
        plus4 = _ring(r + 4)
        minus4 = _ring(r - 4)

        gather_ref[0] = x_ref[...].astype(jnp.bfloat16)
        w_bf[...] = w_ref[...].astype(jnp.bfloat16)

        barrier_sem = pltpu.get_barrier_semaphore()
        for nbr in (left, right, chord, plus4, minus4):
            pl.semaphore_signal(
                barrier_sem, inc=1,
                device_id=(nbr,), device_id_type=pl.DeviceIdType.MESH,
            )
        pl.semaphore_wait(barrier_sem, 5)

        dev_of = {"R": right, "L": left, "C": chord, "P": plus4, "M": minus4}

        def mk(src_slot, dst_slot, seg, send_idx, dev):
            return pltpu.make_async_remote_copy(
                src_ref=gather_ref.at[src_slot, pl.ds(seg * m_seg, m_seg)],
                dst_ref=gather_ref.at[dst_slot, pl.ds(seg * m_seg, m_seg)],
                send_sem=send_sems.at[send_idx, seg],
                recv_sem=recv_sems.at[dst_slot, seg],
                device_id=(dev,),
                device_id_type=pl.DeviceIdType.MESH,
            )

        def compute_block(j):
            if j == 0:
                origin = my
            elif j <= 8:
                origin = _ring(r - j)
            else:
                origin = _ring(r + (j - 8))
            y = jnp.dot(
                gather_ref[j], w_bf[...],
                preferred_element_type=jnp.float32,
            )
            out_ref[pl.ds(origin * m_per, m_per), :] = y * jax.nn.sigmoid(y)

        started = []
        send_idx = 0
        for stage in _SCHEDULE:
            base = send_idx
            for s in range(N_SEG):
                idx = base
                for wait_slot, sends in stage:
                    if wait_slot is not None:
                        mk(wait_slot, wait_slot, s, 0, right).wait_recv()
                    for si, (src, dst, dv) in enumerate(sends):
                        d = mk(src, dst, s, idx + si, dev_of[dv])
                        d.start()
                        started.append(d)
                    idx += len(sends)
            send_idx = idx
            for wait_slot, _ in stage:
                compute_block(0 if wait_slot is None else wait_slot)
        for d in started:
            d.wait_send()

    n_sends = sum(len(s) for stage in _SCHEDULE for _, s in stage)
    return pl.pallas_call(
        body,
        out_shape=jax.ShapeDtypeStruct((N_DEV * m_per, n_per), jnp.float32),
        in_specs=[
            pl.BlockSpec(memory_space=pltpu.VMEM),
            pl.BlockSpec(memory_space=pltpu.VMEM),
        ],
        out_specs=pl.BlockSpec(memory_space=pltpu.VMEM),
        scratch_shapes=[
            pltpu.VMEM((N_DEV, m_per, k), jnp.bfloat16),
            pltpu.VMEM((k, n_per), jnp.bfloat16),
            pltpu.SemaphoreType.DMA((n_sends, N_SEG)),
            pltpu.SemaphoreType.DMA((N_DEV, N_SEG)),
        ],
        compiler_params=pltpu.CompilerParams(collective_id=0),
    )(x, w_mat)


# device time: 20484 ns/iter; 1.0859x vs baseline; 1.0859x over previous
import jax
import jax.numpy as jnp
from jax import lax
from jax.experimental import pallas as pl
from jax.experimental.pallas import tpu as pltpu

N_DEV = 16
N_SEG = 4


def _ring(t):
    t = lax.rem(t + 2 * N_DEV, N_DEV)
    q = t // 4
    j = t % 4
    return jnp.where(
        q == 0, 4 * j,
        jnp.where(q == 1, 15 - 4 * j,
                  jnp.where(q == 2, 4 * j + 2, 13 - 4 * j)),
    )


def _pos(p):
    z = p // 4
    c = p % 4
    return jnp.where(
        c == 0, z,
        jnp.where(c == 1, 15 - z, jnp.where(c == 2, 8 + z, 7 - z)),
    )


_SCHEDULE = (
    ((None, ((0, 8, "C"), (0, 1, "R"), (0, 9, "L"))),),
    ((1, ((1, 2, "R"),)), (9, ((9, 10, "L"),))),
    ((8, ((8, 7, "L"), (8, 15, "R"))),),
    ((2, ((2, 3, "R"),)), (10, ((10, 11, "L"),))),
    ((7, ((7, 6, "L"),)), (15, ((15, 14, "R"),))),
    ((3, ((3, 4, "R"),)), (11, ((11, 12, "L"),))),
    ((6, ((6, 5, "L"),)), (14, ((14, 13, "R"),))),
    ((5, ()), (13, ()), (4, ()), (12, ())),
)


def kernel(x, w_mat):
    m_per, k = x.shape
    _, n_per = w_mat.shape
    m_seg = m_per // N_SEG

    def body(x_ref, w_ref, out_ref, gather_ref, w_bf, send_sems, recv_sems):
        my = lax.axis_index("i")
        r = _pos(my)
        left = _ring(r - 1)
        right = _ring(r + 1)
        chord = _ring(r + 8)

        gather_ref[0] = x_ref[...].astype(jnp.bfloat16)
        w_bf[...] = w_ref[...].astype(jnp.bfloat16)

        barrier_sem = pltpu.get_barrier_semaphore()
        for nbr in (left, right, chord):
            pl.semaphore_signal(
                barrier_sem, inc=1,
                device_id=(nbr,), device_id_type=pl.DeviceIdType.MESH,
            )
        pl.semaphore_wait(barrier_sem, 3)

        dev_of = {"R": right, "L": left, "C": chord}

        def mk(src_slot, dst_slot, seg, send_idx, dev):
            return pltpu.make_async_remote_copy(
                src_ref=gather_ref.at[src_slot, pl.ds(seg * m_seg, m_seg)],
                dst_ref=gather_ref.at[dst_slot, pl.ds(seg * m_seg, m_seg)],
                send_sem=send_sems.at[send_idx, seg],
                recv_sem=recv_sems.at[dst_slot, seg],
                device_id=(dev,),
                device_id_type=pl.DeviceIdType.MESH,
            )

        def compute_block(j):
            if j == 0:
                origin = my
            elif j <= 8:
                origin = _ring(r - j)
            else:
                origin = _ring(r + (j - 8))
            y = jnp.dot(
                gather_ref[j], w_bf[...],
                preferred_element_type=jnp.float32,
            )
            out_ref[pl.ds(origin * m_per, m_per), :] = y * jax.nn.sigmoid(y)

        started = []
        send_idx = 0
        for stage in _SCHEDULE:
            base = send_idx
            for s in range(N_SEG):
                idx = base
                for wait_slot, sends in stage:
                    if wait_slot is not None:
                        mk(wait_slot, wait_slot, s, 0, right).wait_recv()
                    for si, (src, dst, dv) in enumerate(sends):
                        d = mk(src, dst, s, idx + si, dev_of[dv])
                        d.start()
                        started.append(d)
                    idx += len(sends)
            send_idx = idx
            for wait_slot, _ in stage:
                compute_block(0 if wait_slot is None else wait_slot)
        for d in started:
            d.wait_send()

    n_sends = sum(len(s) for stage in _SCHEDULE for _, s in stage)
    return pl.pallas_call(
        body,
        out_shape=jax.ShapeDtypeStruct((N_DEV * m_per, n_per), jnp.float32),
        in_specs=[
            pl.BlockSpec(memory_space=pltpu.VMEM),
            pl.BlockSpec(memory_space=pltpu.VMEM),
        ],
        out_specs=pl.BlockSpec(memory_space=pltpu.VMEM),
        scratch_shapes=[
            pltpu.VMEM((N_DEV, m_per, k), jnp.bfloat16),
            pltpu.VMEM((k, n_per), jnp.bfloat16),
            pltpu.SemaphoreType.DMA((n_sends, N_SEG)),
            pltpu.SemaphoreType.DMA((N_DEV, N_SEG)),
        ],
        compiler_params=pltpu.CompilerParams(collective_id=0),
    )(x, w_mat)
